# baseline (speedup 1.0000x reference)
import functools

import jax
import jax.numpy as jnp
import numpy as np
from jax import lax
from jax.experimental import pallas as pl
from jax.experimental.pallas import tpu as pltpu
from jax.experimental.pallas import tpu_sc as plsc

V = 1000000
BATCH = 16384
FEAT = 64
HID1 = 128
HID2 = 64
PK = FEAT // 2
GROUP = 4
VG = V // GROUP

NC, NS = 2, 16
NW = NC * NS
BPW = BATCH // NW
CHUNK = 128
NCHUNK = BPW // CHUNK

MASK_HI = np.uint32(0xFFFF0000)

CB = 7936
NFULL = V // CB
TAIL = 64
NSTEP = NFULL + 1
NBUF = 3


def _pack_chunk(x):
    xb = x.astype(jnp.bfloat16)
    ii = lax.broadcasted_iota(jnp.int32, (FEAT, FEAT), 0)
    jj = lax.broadcasted_iota(jnp.int32, (FEAT, FEAT), 1)
    ident = jnp.where(ii == jj, 1.0, 0.0).astype(jnp.bfloat16)
    xt = lax.dot_general(xb, ident, (((0,), (0,)), ((), ())),
                         preferred_element_type=jnp.float32)
    a = lax.bitcast_convert_type(xt[:, :PK], jnp.uint32)
    b = lax.bitcast_convert_type(xt[:, PK:], jnp.uint32)
    return (a >> 16) | (b & MASK_HI)


def _pack_body(t_ref, o_ref, in_buf, out_buf, tin, tout, in_sem, out_sem):
    def in_copy(g, slot):
        return pltpu.make_async_copy(
            t_ref.at[:, pl.ds(g * CB, CB)], in_buf.at[slot], in_sem.at[slot])

    def out_copy(g, slot):
        return pltpu.make_async_copy(
            out_buf.at[slot], o_ref.at[pl.ds(g * CB, CB)], out_sem.at[slot])

    g = pl.program_id(0)
    slot = lax.rem(g, NBUF)

    @pl.when(g == 0)
    def _prologue():
        for k in range(NBUF - 1):
            in_copy(k, k).start()

    @pl.when(g < NFULL)
    def _full_step():
        nxt = g + NBUF - 1

        @pl.when(nxt < NFULL)
        def _():
            in_copy(nxt, lax.rem(nxt, NBUF)).start()

        in_copy(g, slot).wait()
        @pl.when(g >= NBUF)
        def _():
            out_copy(g - NBUF, slot).wait()

        out_buf[slot] = _pack_chunk(in_buf[slot])
        out_copy(g, slot).start()

    @pl.when(g == NSTEP - 1)
    def _tail_step():
        for k in range(NBUF):
            gg = NFULL - NBUF + k
            out_copy(gg, gg % NBUF).wait()
        base = NFULL * CB
        pltpu.sync_copy(t_ref.at[:, pl.ds(base, TAIL)], tin)
        tout[...] = _pack_chunk(tin[...])
        pltpu.sync_copy(tout, o_ref.at[pl.ds(base, TAIL)])


_pack = pl.pallas_call(
    _pack_body,
    grid=(NSTEP,),
    in_specs=[pl.BlockSpec(memory_space=pl.ANY)],
    out_specs=pl.BlockSpec(memory_space=pl.ANY),
    out_shape=jax.ShapeDtypeStruct((V, PK), jnp.uint32),
    scratch_shapes=[
        pltpu.VMEM((NBUF, FEAT, CB), jnp.float32),
        pltpu.VMEM((NBUF, CB, PK), jnp.uint32),
        pltpu.VMEM((FEAT, TAIL), jnp.float32),
        pltpu.VMEM((TAIL, PK), jnp.uint32),
        pltpu.SemaphoreType.DMA((NBUF,)),
        pltpu.SemaphoreType.DMA((NBUF,)),
    ],
)


def _gather_body(u_pk4, v_pk4, uq, vq, out_u, out_v,
                 idx_u, idx_v, rows, sem):
    wid = lax.axis_index("s") * NC + lax.axis_index("c")
    row0 = wid * NCHUNK
    base = wid * BPW
    pltpu.sync_copy(uq.at[pl.ds(row0, NCHUNK)], idx_u)
    pltpu.sync_copy(vq.at[pl.ds(row0, NCHUNK)], idx_v)
    for tbl, idx, out in ((u_pk4, idx_u, out_u), (v_pk4, idx_v, out_v)):
        copies = []
        for j in range(NCHUNK):
            copies.append(pltpu.async_copy(
                tbl.at[idx.at[j, 0]], rows.at[pl.ds(j * CHUNK, CHUNK)], sem))
        for c in copies:
            c.wait()
        pltpu.sync_copy(rows, out.at[pl.ds(base, BPW)])


@functools.cache
def _make_gather():
    return pl.kernel(
        _gather_body,
        out_type=(jax.ShapeDtypeStruct((BATCH, 128), jnp.uint32),
                  jax.ShapeDtypeStruct((BATCH, 128), jnp.uint32)),
        mesh=plsc.VectorSubcoreMesh(core_axis_name="c", subcore_axis_name="s"),
        scratch_types=[
            pltpu.VMEM((NCHUNK, 1, CHUNK), jnp.int32),
            pltpu.VMEM((NCHUNK, 1, CHUNK), jnp.int32),
            pltpu.VMEM((BPW, 128), jnp.uint32),
            pltpu.SemaphoreType.DMA,
        ],
    )


BLK = 1024


def _select_unpack(w4, r):
    s = [w4[:, k * PK:(k + 1) * PK] for k in range(GROUP)]
    w = jnp.where(r == 0, s[0],
                  jnp.where(r == 1, s[1], jnp.where(r == 2, s[2], s[3])))
    lo = lax.bitcast_convert_type(w << 16, jnp.float32)
    hi = lax.bitcast_convert_type(w & MASK_HI, jnp.float32)
    return lo, hi


def _mlp_body(u_ref, v_ref, ur_ref, vr_ref, w1a_ref, w1b_ref, w1c_ref,
              w1d_ref, b1_ref, w2_ref, b2_ref, w3_ref, b3_ref, o_ref):
    ulo, uhi = _select_unpack(u_ref[...], ur_ref[...])
    vlo, vhi = _select_unpack(v_ref[...], vr_ref[...])
    h = jnp.dot(ulo, w1a_ref[...], preferred_element_type=jnp.float32)
    h = h + jnp.dot(uhi, w1b_ref[...], preferred_element_type=jnp.float32)
    h = h + jnp.dot(vlo, w1c_ref[...], preferred_element_type=jnp.float32)
    h = h + jnp.dot(vhi, w1d_ref[...], preferred_element_type=jnp.float32)
    h = jnp.maximum(h + b1_ref[...], 0.0)
    h = jnp.maximum(
        jnp.dot(h, w2_ref[...], preferred_element_type=jnp.float32) + b2_ref[...],
        0.0)
    z = jnp.dot(h, w3_ref[...], preferred_element_type=jnp.float32) + b3_ref[...]
    o_ref[...] = jax.nn.sigmoid(z)


_mlp = pl.pallas_call(
    _mlp_body,
    grid=(BATCH // BLK,),
    in_specs=[
        pl.BlockSpec((BLK, 128), lambda i: (i, 0)),
        pl.BlockSpec((BLK, 128), lambda i: (i, 0)),
        pl.BlockSpec((BLK, 1), lambda i: (i, 0)),
        pl.BlockSpec((BLK, 1), lambda i: (i, 0)),
        pl.BlockSpec((PK, HID1), lambda i: (0, 0)),
        pl.BlockSpec((PK, HID1), lambda i: (0, 0)),
        pl.BlockSpec((PK, HID1), lambda i: (0, 0)),
        pl.BlockSpec((PK, HID1), lambda i: (0, 0)),
        pl.BlockSpec((1, HID1), lambda i: (0, 0)),
        pl.BlockSpec((HID1, HID2), lambda i: (0, 0)),
        pl.BlockSpec((1, HID2), lambda i: (0, 0)),
        pl.BlockSpec((HID2, 1), lambda i: (0, 0)),
        pl.BlockSpec((1, 1), lambda i: (0, 0)),
    ],
    out_specs=pl.BlockSpec((BLK, 1), lambda i: (i, 0)),
    out_shape=jax.ShapeDtypeStruct((BATCH, 1), jnp.float32),
)


def kernel(user_table, video_table, W1, b1, W2, b2, W3, b3,
           user_indices, video_indices):
    u_pk4 = _pack(user_table.T).reshape(VG, 128)
    v_pk4 = _pack(video_table.T).reshape(VG, 128)
    ui = user_indices.astype(jnp.int32)
    vi = video_indices.astype(jnp.int32)
    uq = (ui >> 2).reshape(BATCH // CHUNK, 1, CHUNK)
    vq = (vi >> 2).reshape(BATCH // CHUNK, 1, CHUNK)
    gu, gv = _make_gather()(u_pk4, v_pk4, uq, vq)
    return _mlp(gu, gv, (ui & 3).reshape(BATCH, 1), (vi & 3).reshape(BATCH, 1),
                W1[:PK], W1[PK:FEAT], W1[FEAT:FEAT + PK], W1[FEAT + PK:],
                b1.reshape(1, HID1), W2, b2.reshape(1, HID2), W3,
                b3.reshape(1, 1))


# speedup vs baseline: 1.0838x; 1.0838x over previous
import functools

import jax
import jax.numpy as jnp
from jax import lax
from jax.experimental import pallas as pl
from jax.experimental.pallas import tpu as pltpu
from jax.experimental.pallas import tpu_sc as plsc

BATCH = 16384
FEAT = 64
HID1 = 128
HID2 = 64

NC, NS = 2, 16
NW = NC * NS
BPW = BATCH // NW
CHUNK = 128
NCHUNK = BPW // CHUNK


def _gather_body(user_table, video_table, uidx, vidx, out_u, out_v,
                 idx_u, idx_v, rows_u, rows_v, sem):
    wid = lax.axis_index("s") * NC + lax.axis_index("c")
    row0 = wid * NCHUNK
    base = wid * BPW
    pltpu.sync_copy(uidx.at[pl.ds(row0, NCHUNK)], idx_u)
    pltpu.sync_copy(vidx.at[pl.ds(row0, NCHUNK)], idx_v)
    copies = []
    for j in range(NCHUNK):
        copies.append(pltpu.async_copy(
            user_table.at[idx_u.at[j]], rows_u.at[pl.ds(j * CHUNK, CHUNK)], sem))
        copies.append(pltpu.async_copy(
            video_table.at[idx_v.at[j]], rows_v.at[pl.ds(j * CHUNK, CHUNK)], sem))
    for c in copies:
        c.wait()
    pltpu.sync_copy(rows_u, out_u.at[pl.ds(base, BPW)])
    pltpu.sync_copy(rows_v, out_v.at[pl.ds(base, BPW)])


_gather = pl.kernel(
    _gather_body,
    out_type=(jax.ShapeDtypeStruct((BATCH, FEAT), jnp.float32),
              jax.ShapeDtypeStruct((BATCH, FEAT), jnp.float32)),
    mesh=plsc.VectorSubcoreMesh(core_axis_name="c", subcore_axis_name="s"),
    scratch_types=[
        pltpu.VMEM((NCHUNK, CHUNK), jnp.int32),
        pltpu.VMEM((NCHUNK, CHUNK), jnp.int32),
        pltpu.VMEM((BPW, FEAT), jnp.float32),
        pltpu.VMEM((BPW, FEAT), jnp.float32),
        pltpu.SemaphoreType.DMA,
    ],
    compiler_params=pltpu.CompilerParams(use_tc_tiling_on_sc=False),
)


BLK = 1024


def _mlp_body(u_ref, v_ref, w1u_ref, w1v_ref, b1_ref, w2_ref, b2_ref,
              w3_ref, b3_ref, o_ref):
    h = jnp.dot(u_ref[...], w1u_ref[...], preferred_element_type=jnp.float32)
    h = h + jnp.dot(v_ref[...], w1v_ref[...], preferred_element_type=jnp.float32)
    h = jnp.maximum(h + b1_ref[...], 0.0)
    h = jnp.maximum(
        jnp.dot(h, w2_ref[...], preferred_element_type=jnp.float32) + b2_ref[...],
        0.0)
    z = jnp.dot(h, w3_ref[...], preferred_element_type=jnp.float32) + b3_ref[...]
    o_ref[...] = jax.nn.sigmoid(z)


_mlp = pl.pallas_call(
    _mlp_body,
    grid=(BATCH // BLK,),
    in_specs=[
        pl.BlockSpec((BLK, FEAT), lambda i: (i, 0)),
        pl.BlockSpec((BLK, FEAT), lambda i: (i, 0)),
        pl.BlockSpec((FEAT, HID1), lambda i: (0, 0)),
        pl.BlockSpec((FEAT, HID1), lambda i: (0, 0)),
        pl.BlockSpec((1, HID1), lambda i: (0, 0)),
        pl.BlockSpec((HID1, HID2), lambda i: (0, 0)),
        pl.BlockSpec((1, HID2), lambda i: (0, 0)),
        pl.BlockSpec((HID2, 1), lambda i: (0, 0)),
        pl.BlockSpec((1, 1), lambda i: (0, 0)),
    ],
    out_specs=pl.BlockSpec((BLK, 1), lambda i: (i, 0)),
    out_shape=jax.ShapeDtypeStruct((BATCH, 1), jnp.float32),
)


def kernel(user_table, video_table, W1, b1, W2, b2, W3, b3,
           user_indices, video_indices):
    uidx = user_indices.astype(jnp.int32).reshape(BATCH // CHUNK, CHUNK)
    vidx = video_indices.astype(jnp.int32).reshape(BATCH // CHUNK, CHUNK)
    u_vec, v_vec = _gather(user_table, video_table, uidx, vidx)
    return _mlp(u_vec, v_vec, W1[:FEAT], W1[FEAT:], b1.reshape(1, HID1),
                W2, b2.reshape(1, HID2), W3, b3.reshape(1, 1))
